# baseline (speedup 1.0000x reference)
import functools

import jax
import jax.numpy as jnp
from jax import lax
from jax.experimental import pallas as pl
from jax.experimental.pallas import tpu as pltpu
from jax.experimental.pallas import tpu_sc as plsc

T, D, E, TOP_K = 4096, 768, 8, 2
P = T * TOP_K
BLK = 256
NBLK = (P + E * BLK) // BLK
CCAP = NBLK * BLK
BT = 512
LANES = 128

NC, NS, L = 2, 16, 16
NW = NC * NS
PAIRS_PER_W = P // NW
CH_B = 64
TOK_PER_W = T // NW
CH_D = 32
EIDB_LEN = 48


def _router_body(x_ref, wr_ref, br_ref, meta_ref, counts_ref, loss_ref, acc_ref):
    pid = pl.program_id(0)
    nsteps = pl.num_programs(0)

    x = x_ref[...]
    logits = jnp.dot(x, wr_ref[...], preferred_element_type=jnp.float32)
    logits = logits + br_ref[...]
    lane = lax.broadcasted_iota(jnp.int32, (BT, LANES), 1)
    valid = lane < E
    neg = jnp.float32(-1e30)
    logits = jnp.where(valid, logits, neg)

    m = jnp.max(logits, axis=-1, keepdims=True)
    ex = jnp.exp(logits - m)
    ex = jnp.where(valid, ex, 0.0)
    probs = ex / jnp.sum(ex, axis=-1, keepdims=True)

    big = jnp.int32(10**9)
    m1 = jnp.max(probs, axis=-1, keepdims=True)
    i1 = jnp.min(jnp.where((probs == m1) & valid, lane, big), axis=-1, keepdims=True)
    probs2 = jnp.where(lane == i1, neg, probs)
    m2 = jnp.max(probs2, axis=-1, keepdims=True)
    i2 = jnp.min(jnp.where((probs2 == m2) & valid, lane, big), axis=-1, keepdims=True)

    oh1 = (lane == i1).astype(jnp.float32)
    oh2 = (lane == i2).astype(jnp.float32)
    wsum = m1 + m2
    g1 = m1 / wsum
    g2 = m2 / wsum

    @pl.when(pid == 0)
    def _init():
        acc_ref[...] = jnp.zeros_like(acc_ref)

    carry = acc_ref[0:1, :]

    r_io = lax.broadcasted_iota(jnp.int32, (BT, BT), 0)
    c_io = lax.broadcasted_iota(jnp.int32, (BT, BT), 1)
    lstrict = (c_io < r_io).astype(jnp.float32)
    p1 = jnp.dot(lstrict, oh1, preferred_element_type=jnp.float32)
    rank1 = jnp.sum(p1 * oh1, axis=-1, keepdims=True) + jnp.sum(
        carry * oh1, axis=-1, keepdims=True)
    carry1 = carry + jnp.sum(oh1, axis=0, keepdims=True)
    p2 = jnp.dot(lstrict, oh2, preferred_element_type=jnp.float32)
    rank2 = jnp.sum(p2 * oh2, axis=-1, keepdims=True) + jnp.sum(
        carry1 * oh2, axis=-1, keepdims=True)
    acc_ref[0:1, :] = carry1 + jnp.sum(oh2, axis=0, keepdims=True)

    acc_ref[1:2, :] += jnp.sum(probs, axis=0, keepdims=True)

    meta = (
        rank1 * (lane == 0)
        + rank2 * (lane == 1)
        + i1.astype(jnp.float32) * (lane == 2)
        + i2.astype(jnp.float32) * (lane == 3)
        + g1 * (lane == 4)
        + g2 * (lane == 5)
    )
    meta_ref[...] = meta

    @pl.when(pid == nsteps - 1)
    def _fin():
        row = lax.broadcasted_iota(jnp.int32, (8, LANES), 0)
        counts_ref[...] = jnp.where(
            row == 0, jnp.broadcast_to(acc_ref[0:1, :], (8, LANES)), 0.0)
        f = acc_ref[0:1, :] / jnp.float32(T)
        pm = acc_ref[1:2, :] / jnp.float32(T)
        loss_ref[...] = jnp.sum(jnp.float32(E) * f * pm).reshape(1, 1)


def _run_router(x, wr_pad, br_pad):
    return pl.pallas_call(
        _router_body,
        grid=(T // BT,),
        in_specs=[
            pl.BlockSpec((BT, D), lambda i: (i, 0)),
            pl.BlockSpec((D, LANES), lambda i: (0, 0)),
            pl.BlockSpec((1, LANES), lambda i: (0, 0)),
        ],
        out_specs=[
            pl.BlockSpec((BT, LANES), lambda i: (i, 0)),
            pl.BlockSpec((8, LANES), lambda i: (0, 0)),
            pl.BlockSpec((1, 1), lambda i: (0, 0)),
        ],
        out_shape=[
            jax.ShapeDtypeStruct((T, LANES), jnp.float32),
            jax.ShapeDtypeStruct((8, LANES), jnp.float32),
            jax.ShapeDtypeStruct((1, 1), jnp.float32),
        ],
        scratch_shapes=[pltpu.VMEM((2, LANES), jnp.float32)],
    )(x, wr_pad, br_pad)


def _load_offsets(counts_hbm, cnt_vm, off_vm, ncum_vm):
    pltpu.sync_copy(counts_hbm.at[0, pl.ds(0, 16)], cnt_vm)
    cnt = cnt_vm[...].astype(jnp.int32)
    lane = lax.iota(jnp.int32, 16)
    zero = jnp.zeros((16,), jnp.int32)
    nb = lax.shift_right_logical(cnt + (BLK - 1), 8)
    nb = jnp.where(lane < E, nb, zero)
    ncum = plsc.cumsum(nb)
    off_vm[...] = lax.shift_left(ncum - nb, 8)
    return ncum


def _scatter_body(meta_hbm, counts_hbm, x_hbm, xs_hbm, eidb_hbm,
                  cnt_vm, off_vm, ncum_vm, metabuf, xbuf, idxbuf, eidb_vm, sem):
    wid = lax.axis_index("s") * NC + lax.axis_index("c")
    lane = lax.iota(jnp.int32, 16)

    ncum = _load_offsets(counts_hbm, cnt_vm, off_vm, ncum_vm)

    @pl.when(wid == 0)
    def _eidb():
        seven = jnp.full((16,), 7, jnp.int32)
        zero16 = jnp.zeros((16,), jnp.int32)
        for c in range(EIDB_LEN // 16):
            bvec = lane + 16 * c
            acc = jnp.zeros((16,), jnp.int32)
            for e in range(E):
                th = jnp.sum(jnp.where(lane == e, ncum, zero16))
                acc = acc + (bvec >= th).astype(jnp.int32)
            eidb_vm[pl.ds(16 * c, 16)] = jnp.minimum(acc, seven)
        nbu = jnp.sum(jnp.where(lane == E - 1, ncum, zero16))
        plsc.store_scatter(eidb_vm, [jnp.full((16,), NBLK, jnp.int32)],
                           zero16 + nbu, mask=lane == 0)
        pltpu.sync_copy(eidb_vm, eidb_hbm)

    base = wid * PAIRS_PER_W
    slot = base // T
    tbase = base % T
    for k in range(PAIRS_PER_W // CH_B):
        t0 = tbase + k * CH_B
        pltpu.sync_copy(meta_hbm.at[pl.ds(t0, CH_B)], metabuf)
        pltpu.sync_copy(x_hbm.at[pl.ds(t0, CH_B)], xbuf)
        for i in range(CH_B // 16):
            tok = lane + 16 * i
            col_r = jnp.full((16,), 0, jnp.int32) + slot
            col_e = jnp.full((16,), 2, jnp.int32) + slot
            rank = plsc.load_gather(metabuf, [tok, col_r])
            eid = plsc.load_gather(metabuf, [tok, col_e]).astype(jnp.int32)
            off = plsc.load_gather(off_vm, [eid])
            idxbuf[pl.ds(16 * i, 16)] = off + rank.astype(jnp.int32)
        pltpu.async_copy(xbuf, xs_hbm.at[idxbuf], sem).wait()


def _run_scatter(meta, counts, x):
    mesh = plsc.VectorSubcoreMesh(core_axis_name="c", subcore_axis_name="s")
    kfn = pl.kernel(
        _scatter_body,
        compiler_params=pltpu.CompilerParams(needs_layout_passes=False),
        out_type=[
            jax.ShapeDtypeStruct((CCAP, D), jnp.float32),
            jax.ShapeDtypeStruct((EIDB_LEN,), jnp.int32),
        ],
        mesh=mesh,
        scratch_types=[
            pltpu.VMEM((16,), jnp.float32),
            pltpu.VMEM((16,), jnp.int32),
            pltpu.VMEM((16,), jnp.int32),
            pltpu.VMEM((CH_B, LANES), jnp.float32),
            pltpu.VMEM((CH_B, D), jnp.float32),
            pltpu.VMEM((CH_B,), jnp.int32),
            pltpu.VMEM((EIDB_LEN,), jnp.int32),
            pltpu.SemaphoreType.DMA,
        ],
    )
    return kfn(meta, counts, x)


def _gmm_body(eidb_ref, xs_ref, we_ref, be_ref, ys_ref):
    pid = pl.program_id(0)

    @pl.when(pid < eidb_ref[NBLK])
    def _go():
        ys_ref[...] = (
            jnp.dot(xs_ref[...], we_ref[0], preferred_element_type=jnp.float32)
            + be_ref[0]
        )


def _run_gmm(eidb, xs, We, be):
    grid_spec = pltpu.PrefetchScalarGridSpec(
        num_scalar_prefetch=1,
        grid=(NBLK,),
        in_specs=[
            pl.BlockSpec((BLK, D), lambda i, eidb: (i, 0)),
            pl.BlockSpec((1, D, D), lambda i, eidb: (eidb[i], 0, 0)),
            pl.BlockSpec((1, 1, D), lambda i, eidb: (eidb[i], 0, 0)),
        ],
        out_specs=pl.BlockSpec((BLK, D), lambda i, eidb: (i, 0)),
    )
    return pl.pallas_call(
        _gmm_body,
        grid_spec=grid_spec,
        out_shape=jax.ShapeDtypeStruct((CCAP, D), jnp.float32),
    )(eidb, xs, We, be.reshape(E, 1, D))


def _combine_body(meta_hbm, counts_hbm, ys_hbm, out_hbm,
                  cnt_vm, off_vm, ncum_vm, metabuf, y0buf, y1buf, obuf,
                  idx0, idx1, g0vm, g1vm, sem0, sem1):
    wid = lax.axis_index("s") * NC + lax.axis_index("c")
    lane = lax.iota(jnp.int32, 16)

    _load_offsets(counts_hbm, cnt_vm, off_vm, ncum_vm)
    tbase = wid * TOK_PER_W
    for k in range(TOK_PER_W // CH_D):
        t0 = tbase + k * CH_D
        pltpu.sync_copy(meta_hbm.at[pl.ds(t0, CH_D)], metabuf)
        for i in range(CH_D // 16):
            tok = lane + 16 * i
            rank1 = plsc.load_gather(metabuf, [tok, jnp.full((16,), 0, jnp.int32)])
            rank2 = plsc.load_gather(metabuf, [tok, jnp.full((16,), 1, jnp.int32)])
            eid1 = plsc.load_gather(
                metabuf, [tok, jnp.full((16,), 2, jnp.int32)]).astype(jnp.int32)
            eid2 = plsc.load_gather(
                metabuf, [tok, jnp.full((16,), 3, jnp.int32)]).astype(jnp.int32)
            idx0[pl.ds(16 * i, 16)] = (
                plsc.load_gather(off_vm, [eid1]) + rank1.astype(jnp.int32))
            idx1[pl.ds(16 * i, 16)] = (
                plsc.load_gather(off_vm, [eid2]) + rank2.astype(jnp.int32))
            g0vm[pl.ds(16 * i, 16)] = plsc.load_gather(
                metabuf, [tok, jnp.full((16,), 4, jnp.int32)])
            g1vm[pl.ds(16 * i, 16)] = plsc.load_gather(
                metabuf, [tok, jnp.full((16,), 5, jnp.int32)])
        cp0 = pltpu.async_copy(ys_hbm.at[idx0], y0buf, sem0)
        cp1 = pltpu.async_copy(ys_hbm.at[idx1], y1buf, sem1)
        cp0.wait()
        cp1.wait()

        def _tok(i, carry):
            row = jnp.zeros((16,), jnp.int32) + i
            g0 = plsc.load_gather(g0vm, [row])
            g1 = plsc.load_gather(g1vm, [row])
            for c in range(D // 16):
                colv = lane + 16 * c
                y0 = plsc.load_gather(y0buf, [row, colv])
                y1 = plsc.load_gather(y1buf, [row, colv])
                plsc.store_scatter(obuf, [row, colv], g0 * y0 + g1 * y1)
            return carry

        lax.fori_loop(0, CH_D, _tok, 0)
        pltpu.sync_copy(obuf, out_hbm.at[pl.ds(t0, CH_D)])


def _run_combine(meta, counts, ys):
    mesh = plsc.VectorSubcoreMesh(core_axis_name="c", subcore_axis_name="s")
    kfn = pl.kernel(
        _combine_body,
        compiler_params=pltpu.CompilerParams(needs_layout_passes=False),
        out_type=jax.ShapeDtypeStruct((T, D), jnp.float32),
        mesh=mesh,
        scratch_types=[
            pltpu.VMEM((16,), jnp.float32),
            pltpu.VMEM((16,), jnp.int32),
            pltpu.VMEM((16,), jnp.int32),
            pltpu.VMEM((CH_D, LANES), jnp.float32),
            pltpu.VMEM((CH_D, D), jnp.float32),
            pltpu.VMEM((CH_D, D), jnp.float32),
            pltpu.VMEM((CH_D, D), jnp.float32),
            pltpu.VMEM((CH_D,), jnp.int32),
            pltpu.VMEM((CH_D,), jnp.int32),
            pltpu.VMEM((CH_D,), jnp.float32),
            pltpu.VMEM((CH_D,), jnp.float32),
            pltpu.SemaphoreType.DMA,
            pltpu.SemaphoreType.DMA,
        ],
    )
    return kfn(meta, counts, ys)


def kernel(x, Wr, br, We, be):
    wr_pad = jnp.zeros((D, LANES), jnp.float32).at[:, :E].set(Wr)
    br_pad = jnp.zeros((1, LANES), jnp.float32).at[0, :E].set(br)

    meta, counts, loss = _run_router(x, wr_pad, br_pad)
    xs, eidb = _run_scatter(meta, counts, x)
    ys = _run_gmm(eidb, xs, We, be)
    out = _run_combine(meta, counts, ys)
    return out, loss.reshape(())


# speedup vs baseline: 1.1583x; 1.1583x over previous
import functools

import jax
import jax.numpy as jnp
from jax import lax
from jax.experimental import pallas as pl
from jax.experimental.pallas import tpu as pltpu
from jax.experimental.pallas import tpu_sc as plsc

T, D, E, TOP_K = 4096, 768, 8, 2
P = T * TOP_K
BLK = 256
NBLK = (P + E * BLK) // BLK
CCAP = NBLK * BLK
BT = 512
LANES = 128

NC, NS, L = 2, 16, 16
NW = NC * NS
PAIRS_PER_W = P // NW
CH_B = 32
TOK_PER_W = T // NW
CH_D = 16
EIDB_LEN = 48


def _router_body(x_ref, wr_ref, br_ref, meta_ref, counts_ref, loss_ref, acc_ref):
    pid = pl.program_id(0)
    nsteps = pl.num_programs(0)

    x = x_ref[...]
    logits = jnp.dot(x, wr_ref[...], preferred_element_type=jnp.float32)
    logits = logits + br_ref[...]
    lane = lax.broadcasted_iota(jnp.int32, (BT, LANES), 1)
    valid = lane < E
    neg = jnp.float32(-1e30)
    logits = jnp.where(valid, logits, neg)

    m = jnp.max(logits, axis=-1, keepdims=True)
    ex = jnp.exp(logits - m)
    ex = jnp.where(valid, ex, 0.0)
    probs = ex / jnp.sum(ex, axis=-1, keepdims=True)

    big = jnp.int32(10**9)
    m1 = jnp.max(probs, axis=-1, keepdims=True)
    i1 = jnp.min(jnp.where((probs == m1) & valid, lane, big), axis=-1, keepdims=True)
    probs2 = jnp.where(lane == i1, neg, probs)
    m2 = jnp.max(probs2, axis=-1, keepdims=True)
    i2 = jnp.min(jnp.where((probs2 == m2) & valid, lane, big), axis=-1, keepdims=True)

    oh1 = (lane == i1).astype(jnp.float32)
    oh2 = (lane == i2).astype(jnp.float32)
    wsum = m1 + m2
    g1 = m1 / wsum
    g2 = m2 / wsum

    @pl.when(pid == 0)
    def _init():
        acc_ref[...] = jnp.zeros_like(acc_ref)

    carry = acc_ref[0:1, :]

    r_io = lax.broadcasted_iota(jnp.int32, (BT, BT), 0)
    c_io = lax.broadcasted_iota(jnp.int32, (BT, BT), 1)
    lstrict = (c_io < r_io).astype(jnp.float32)
    p1 = jnp.dot(lstrict, oh1, preferred_element_type=jnp.float32)
    rank1 = jnp.sum(p1 * oh1, axis=-1, keepdims=True) + jnp.sum(
        carry * oh1, axis=-1, keepdims=True)
    carry1 = carry + jnp.sum(oh1, axis=0, keepdims=True)
    p2 = jnp.dot(lstrict, oh2, preferred_element_type=jnp.float32)
    rank2 = jnp.sum(p2 * oh2, axis=-1, keepdims=True) + jnp.sum(
        carry1 * oh2, axis=-1, keepdims=True)
    acc_ref[0:1, :] = carry1 + jnp.sum(oh2, axis=0, keepdims=True)

    acc_ref[1:2, :] += jnp.sum(probs, axis=0, keepdims=True)

    meta = (
        rank1 * (lane == 0)
        + rank2 * (lane == 1)
        + i1.astype(jnp.float32) * (lane == 2)
        + i2.astype(jnp.float32) * (lane == 3)
        + g1 * (lane == 4)
        + g2 * (lane == 5)
    )
    meta_ref[...] = meta

    @pl.when(pid == nsteps - 1)
    def _fin():
        row = lax.broadcasted_iota(jnp.int32, (8, LANES), 0)
        counts_ref[...] = jnp.where(
            row == 0, jnp.broadcast_to(acc_ref[0:1, :], (8, LANES)), 0.0)
        f = acc_ref[0:1, :] / jnp.float32(T)
        pm = acc_ref[1:2, :] / jnp.float32(T)
        loss_ref[...] = jnp.sum(jnp.float32(E) * f * pm).reshape(1, 1)


def _run_router(x, wr_pad, br_pad):
    return pl.pallas_call(
        _router_body,
        grid=(T // BT,),
        in_specs=[
            pl.BlockSpec((BT, D), lambda i: (i, 0)),
            pl.BlockSpec((D, LANES), lambda i: (0, 0)),
            pl.BlockSpec((1, LANES), lambda i: (0, 0)),
        ],
        out_specs=[
            pl.BlockSpec((BT, LANES), lambda i: (i, 0)),
            pl.BlockSpec((8, LANES), lambda i: (0, 0)),
            pl.BlockSpec((1, 1), lambda i: (0, 0)),
        ],
        out_shape=[
            jax.ShapeDtypeStruct((T, LANES), jnp.float32),
            jax.ShapeDtypeStruct((8, LANES), jnp.float32),
            jax.ShapeDtypeStruct((1, 1), jnp.float32),
        ],
        scratch_shapes=[pltpu.VMEM((2, LANES), jnp.float32)],
    )(x, wr_pad, br_pad)


def _load_offsets(counts_hbm, cnt_vm, off_vm, ncum_vm):
    pltpu.sync_copy(counts_hbm.at[0, pl.ds(0, 16)], cnt_vm)
    cnt = cnt_vm[...].astype(jnp.int32)
    lane = lax.iota(jnp.int32, 16)
    zero = jnp.zeros((16,), jnp.int32)
    nb = lax.shift_right_logical(cnt + (BLK - 1), 8)
    nb = jnp.where(lane < E, nb, zero)
    ncum = plsc.cumsum(nb)
    off_vm[...] = lax.shift_left(ncum - nb, 8)
    return ncum


def _scatter_body(meta_hbm, counts_hbm, x_hbm, xs_hbm, eidb_hbm,
                  cnt_vm, off_vm, ncum_vm, metabuf0, xbuf0, idxbuf0, sem_a,
                  metabuf1, xbuf1, idxbuf1, sem_b, eidb_vm):
    wid = lax.axis_index("s") * NC + lax.axis_index("c")
    lane = lax.iota(jnp.int32, 16)
    bufs = [(metabuf0, xbuf0, idxbuf0, sem_a), (metabuf1, xbuf1, idxbuf1, sem_b)]

    ncum = _load_offsets(counts_hbm, cnt_vm, off_vm, ncum_vm)

    @pl.when(wid == 0)
    def _eidb():
        seven = jnp.full((16,), 7, jnp.int32)
        zero16 = jnp.zeros((16,), jnp.int32)
        for c in range(EIDB_LEN // 16):
            bvec = lane + 16 * c
            acc = jnp.zeros((16,), jnp.int32)
            for e in range(E):
                th = jnp.sum(jnp.where(lane == e, ncum, zero16))
                acc = acc + (bvec >= th).astype(jnp.int32)
            eidb_vm[pl.ds(16 * c, 16)] = jnp.minimum(acc, seven)
        nbu = jnp.sum(jnp.where(lane == E - 1, ncum, zero16))
        plsc.store_scatter(eidb_vm, [jnp.full((16,), NBLK, jnp.int32)],
                           zero16 + nbu, mask=lane == 0)
        pltpu.sync_copy(eidb_vm, eidb_hbm)

    base = wid * PAIRS_PER_W
    slot = base // T
    tbase = base % T
    nchunk = PAIRS_PER_W // CH_B
    cps = [None, None]
    for k in range(nchunk):
        b = k % 2
        mb, xb, ib, sm = bufs[b]
        if cps[b] is not None:
            cps[b].wait()
        t0 = tbase + k * CH_B
        pltpu.sync_copy(meta_hbm.at[pl.ds(t0, CH_B)], mb)
        pltpu.sync_copy(x_hbm.at[pl.ds(t0, CH_B)], xb)
        for i in range(CH_B // 16):
            tok = lane + 16 * i
            col_r = jnp.full((16,), 0, jnp.int32) + slot
            col_e = jnp.full((16,), 2, jnp.int32) + slot
            rank = plsc.load_gather(mb, [tok, col_r])
            eid = plsc.load_gather(mb, [tok, col_e]).astype(jnp.int32)
            off = plsc.load_gather(off_vm, [eid])
            ib[pl.ds(16 * i, 16)] = off + rank.astype(jnp.int32)
        cps[b] = pltpu.async_copy(xb, xs_hbm.at[ib], sm)
    for b in range(2):
        if cps[b] is not None:
            cps[b].wait()


def _run_scatter(meta, counts, x):
    mesh = plsc.VectorSubcoreMesh(core_axis_name="c", subcore_axis_name="s")
    kfn = pl.kernel(
        _scatter_body,
        compiler_params=pltpu.CompilerParams(needs_layout_passes=False),
        out_type=[
            jax.ShapeDtypeStruct((CCAP, D), jnp.float32),
            jax.ShapeDtypeStruct((EIDB_LEN,), jnp.int32),
        ],
        mesh=mesh,
        scratch_types=[
            pltpu.VMEM((16,), jnp.float32),
            pltpu.VMEM((16,), jnp.int32),
            pltpu.VMEM((16,), jnp.int32),
            pltpu.VMEM((CH_B, LANES), jnp.float32),
            pltpu.VMEM((CH_B, D), jnp.float32),
            pltpu.VMEM((CH_B,), jnp.int32),
            pltpu.SemaphoreType.DMA,
            pltpu.VMEM((CH_B, LANES), jnp.float32),
            pltpu.VMEM((CH_B, D), jnp.float32),
            pltpu.VMEM((CH_B,), jnp.int32),
            pltpu.SemaphoreType.DMA,
            pltpu.VMEM((EIDB_LEN,), jnp.int32),
        ],
    )
    return kfn(meta, counts, x)


def _gmm_body(eidb_ref, xs_ref, we_ref, be_ref, ys_ref):
    pid = pl.program_id(0)

    @pl.when(pid < eidb_ref[NBLK])
    def _go():
        ys_ref[...] = (
            jnp.dot(xs_ref[...], we_ref[0], preferred_element_type=jnp.float32)
            + be_ref[0]
        )


def _run_gmm(eidb, xs, We, be):
    grid_spec = pltpu.PrefetchScalarGridSpec(
        num_scalar_prefetch=1,
        grid=(NBLK,),
        in_specs=[
            pl.BlockSpec((BLK, D), lambda i, eidb: (i, 0)),
            pl.BlockSpec((1, D, D), lambda i, eidb: (eidb[i], 0, 0)),
            pl.BlockSpec((1, 1, D), lambda i, eidb: (eidb[i], 0, 0)),
        ],
        out_specs=pl.BlockSpec((BLK, D), lambda i, eidb: (i, 0)),
    )
    return pl.pallas_call(
        _gmm_body,
        grid_spec=grid_spec,
        out_shape=jax.ShapeDtypeStruct((CCAP, D), jnp.float32),
    )(eidb, xs, We, be.reshape(E, 1, D))


def _combine_body(meta_hbm, counts_hbm, ys_hbm, out_hbm,
                  cnt_vm, off_vm, ncum_vm, metabuf, y0buf, y1buf, obuf,
                  idx0, idx1, g0vm, g1vm, sem0, sem1):
    wid = lax.axis_index("s") * NC + lax.axis_index("c")
    lane = lax.iota(jnp.int32, 16)

    _load_offsets(counts_hbm, cnt_vm, off_vm, ncum_vm)
    tbase = wid * TOK_PER_W
    for k in range(TOK_PER_W // CH_D):
        t0 = tbase + k * CH_D
        pltpu.sync_copy(meta_hbm.at[pl.ds(t0, CH_D)], metabuf)
        for i in range(CH_D // 16):
            tok = lane + 16 * i
            rank1 = plsc.load_gather(metabuf, [tok, jnp.full((16,), 0, jnp.int32)])
            rank2 = plsc.load_gather(metabuf, [tok, jnp.full((16,), 1, jnp.int32)])
            eid1 = plsc.load_gather(
                metabuf, [tok, jnp.full((16,), 2, jnp.int32)]).astype(jnp.int32)
            eid2 = plsc.load_gather(
                metabuf, [tok, jnp.full((16,), 3, jnp.int32)]).astype(jnp.int32)
            idx0[pl.ds(16 * i, 16)] = (
                plsc.load_gather(off_vm, [eid1]) + rank1.astype(jnp.int32))
            idx1[pl.ds(16 * i, 16)] = (
                plsc.load_gather(off_vm, [eid2]) + rank2.astype(jnp.int32))
            g0vm[pl.ds(16 * i, 16)] = plsc.load_gather(
                metabuf, [tok, jnp.full((16,), 4, jnp.int32)])
            g1vm[pl.ds(16 * i, 16)] = plsc.load_gather(
                metabuf, [tok, jnp.full((16,), 5, jnp.int32)])
        cp0 = pltpu.async_copy(ys_hbm.at[idx0], y0buf, sem0)
        cp1 = pltpu.async_copy(ys_hbm.at[idx1], y1buf, sem1)
        cp0.wait()
        cp1.wait()

        def _tok(i, carry):
            iv = jnp.zeros((16,), jnp.int32) + i
            g0 = plsc.load_gather(g0vm, [iv])
            g1 = plsc.load_gather(g1vm, [iv])
            for c in range(D // 16):
                sl = pl.ds(16 * c, 16)
                obuf[i, sl] = g0 * y0buf[i, sl] + g1 * y1buf[i, sl]
            return carry

        lax.fori_loop(0, CH_D, _tok, 0)
        pltpu.sync_copy(obuf, out_hbm.at[pl.ds(t0, CH_D)])


def _run_combine(meta, counts, ys):
    mesh = plsc.VectorSubcoreMesh(core_axis_name="c", subcore_axis_name="s")
    kfn = pl.kernel(
        _combine_body,
        compiler_params=pltpu.CompilerParams(needs_layout_passes=False),
        out_type=jax.ShapeDtypeStruct((T, D), jnp.float32),
        mesh=mesh,
        scratch_types=[
            pltpu.VMEM((16,), jnp.float32),
            pltpu.VMEM((16,), jnp.int32),
            pltpu.VMEM((16,), jnp.int32),
            pltpu.VMEM((CH_D, LANES), jnp.float32),
            pltpu.VMEM((CH_D, D), jnp.float32),
            pltpu.VMEM((CH_D, D), jnp.float32),
            pltpu.VMEM((CH_D, D), jnp.float32),
            pltpu.VMEM((CH_D,), jnp.int32),
            pltpu.VMEM((CH_D,), jnp.int32),
            pltpu.VMEM((CH_D,), jnp.float32),
            pltpu.VMEM((CH_D,), jnp.float32),
            pltpu.SemaphoreType.DMA,
            pltpu.SemaphoreType.DMA,
        ],
    )
    return kfn(meta, counts, ys)


def kernel(x, Wr, br, We, be):
    wr_pad = jnp.zeros((D, LANES), jnp.float32).at[:, :E].set(Wr)
    br_pad = jnp.zeros((1, LANES), jnp.float32).at[0, :E].set(br)

    meta, counts, loss = _run_router(x, wr_pad, br_pad)
    xs, eidb = _run_scatter(meta, counts, x)
    ys = _run_gmm(eidb, xs, We, be)
    out = _run_combine(meta, counts, ys)
    return out, loss.reshape(())


# speedup vs baseline: 2.8633x; 2.4719x over previous
import jax
import jax.numpy as jnp
from jax.experimental import pallas as pl
from jax.experimental.pallas import tpu as pltpu

T, D, E, TOP_K = 4096, 768, 8, 2
BT = 1024
LANES = 128


def _moe_block(x_ref, wr_ref, br_ref, we_ref, be_ref, out_ref, loss_ref, acc_ref):
    pid = pl.program_id(0)
    nsteps = pl.num_programs(0)

    x = x_ref[...]
    logits = jnp.dot(x, wr_ref[...], preferred_element_type=jnp.float32)
    logits = logits + br_ref[...]
    lane = jax.lax.broadcasted_iota(jnp.int32, (BT, LANES), 1)
    valid = lane < E
    neg = jnp.float32(-1e30)
    logits = jnp.where(valid, logits, neg)

    m = jnp.max(logits, axis=-1, keepdims=True)
    ex = jnp.exp(logits - m)
    ex = jnp.where(valid, ex, 0.0)
    denom = jnp.sum(ex, axis=-1, keepdims=True)
    probs = ex / denom

    big = jnp.int32(10**9)
    m1 = jnp.max(probs, axis=-1, keepdims=True)
    i1 = jnp.min(jnp.where((probs == m1) & valid, lane, big), axis=-1, keepdims=True)
    probs2 = jnp.where(lane == i1, neg, probs)
    m2 = jnp.max(probs2, axis=-1, keepdims=True)
    i2 = jnp.min(jnp.where((probs2 == m2) & valid, lane, big), axis=-1, keepdims=True)

    oh1 = (lane == i1).astype(jnp.float32)
    oh2 = (lane == i2).astype(jnp.float32)
    wsum = m1 + m2
    gate = (m1 / wsum) * oh1 + (m2 / wsum) * oh2

    part = jnp.sum(oh1 + oh2, axis=0, keepdims=True)
    psum = jnp.sum(probs, axis=0, keepdims=True)

    @pl.when(pid == 0)
    def _init():
        acc_ref[...] = jnp.zeros_like(acc_ref)

    acc_ref[0:1, :] += part
    acc_ref[1:2, :] += psum

    acc = jnp.zeros((BT, D), dtype=jnp.float32)
    for e in range(E):
        ye = jnp.dot(x, we_ref[e], preferred_element_type=jnp.float32)
        ye = ye + be_ref[e][None, :]
        acc = acc + gate[:, e][:, None] * ye
    out_ref[...] = acc

    @pl.when(pid == nsteps - 1)
    def _fin():
        f = acc_ref[0:1, :] / jnp.float32(T)
        p = acc_ref[1:2, :] / jnp.float32(T)
        loss_ref[...] = jnp.sum(jnp.float32(E) * f * p).reshape(1, 1)


def kernel(x, Wr, br, We, be):
    wr_pad = jnp.zeros((D, LANES), jnp.float32).at[:, :E].set(Wr)
    br_pad = jnp.zeros((1, LANES), jnp.float32).at[0, :E].set(br)

    grid = (T // BT,)
    out, loss = pl.pallas_call(
        _moe_block,
        grid=grid,
        in_specs=[
            pl.BlockSpec((BT, D), lambda i: (i, 0)),
            pl.BlockSpec((D, LANES), lambda i: (0, 0)),
            pl.BlockSpec((1, LANES), lambda i: (0, 0)),
            pl.BlockSpec((E, D, D), lambda i: (0, 0, 0)),
            pl.BlockSpec((E, D), lambda i: (0, 0)),
        ],
        out_specs=[
            pl.BlockSpec((BT, D), lambda i: (i, 0)),
            pl.BlockSpec((1, 1), lambda i: (0, 0)),
        ],
        out_shape=[
            jax.ShapeDtypeStruct((T, D), jnp.float32),
            jax.ShapeDtypeStruct((1, 1), jnp.float32),
        ],
        scratch_shapes=[pltpu.VMEM((2, LANES), jnp.float32)],
    )(x, wr_pad, br_pad, We, be)
    return out, loss.reshape(())
